# Initial kernel scaffold; baseline (speedup 1.0000x reference)
import jax
import jax.numpy as jnp
from jax.experimental import pallas as pl


def kernel(input, embedding_matrix):
    raise NotImplementedError("write your pallas kernel here")


# speedup vs baseline: 1.0269x; 1.0269x over previous
import functools

import jax
import jax.numpy as jnp
from jax import lax
from jax.experimental import pallas as pl
from jax.experimental.pallas import tpu as pltpu
from jax.experimental.pallas import tpu_sc as plsc

_NC = 2
_NS = 16
_NW = _NC * _NS
_CH = 64


@functools.partial(jax.jit, static_argnums=())
def _sc_embedding_gather(idx_grp, table):
    nw, nch, ch = idx_grp.shape
    d = table.shape[1]
    b = nw * nch * ch
    mesh = plsc.VectorSubcoreMesh(core_axis_name="c", subcore_axis_name="s")

    @functools.partial(
        pl.kernel,
        out_type=jax.ShapeDtypeStruct((b, d), jnp.float32),
        mesh=mesh,
        scratch_types=[
            pltpu.VMEM((nch, ch), jnp.int32),
            pltpu.VMEM((ch, d), jnp.float32),
            pltpu.VMEM((ch, d), jnp.float32),
            pltpu.SemaphoreType.DMA,
        ],
        compiler_params=pltpu.CompilerParams(use_tc_tiling_on_sc=False),
    )
    def k(idx_hbm, table_hbm, out_hbm, idx_v, buf0, buf1, sem):
        wid = lax.axis_index("s") * _NC + lax.axis_index("c")
        base = wid * (nch * ch)
        pltpu.sync_copy(idx_hbm.at[wid], idx_v)
        bufs = (buf0, buf1)
        pending = pltpu.async_copy(table_hbm.at[idx_v.at[0]], buf0, sem)
        for c in range(nch):
            cur = bufs[c % 2]
            pending.wait()
            if c + 1 < nch:
                pending = pltpu.async_copy(
                    table_hbm.at[idx_v.at[c + 1]], bufs[(c + 1) % 2], sem
                )
            pltpu.sync_copy(cur, out_hbm.at[pl.ds(base + c * ch, ch), :])

    return k(idx_grp, table)


def kernel(input, embedding_matrix):
    b = input.size
    d = embedding_matrix.shape[1]
    bpw = b // _NW
    nch = bpw // _CH
    idx_grp = input.reshape(_NW, nch, _CH).astype(jnp.int32)
    out = _sc_embedding_gather(idx_grp, embedding_matrix)
    return out.reshape(input.shape + (d,))


# speedup vs baseline: 1.0276x; 1.0007x over previous
import functools

import jax
import jax.numpy as jnp
from jax import lax
from jax.experimental import pallas as pl
from jax.experimental.pallas import tpu as pltpu
from jax.experimental.pallas import tpu_sc as plsc

_NC = 2
_NS = 16
_NW = _NC * _NS


@jax.jit
def _sc_embedding_gather(idx_grp, table):
    nw, bpw, s = idx_grp.shape
    d = table.shape[1]
    mesh = plsc.VectorSubcoreMesh(core_axis_name="c", subcore_axis_name="s")

    @functools.partial(
        pl.kernel,
        out_type=jax.ShapeDtypeStruct((nw * bpw, s, d), jnp.float32),
        mesh=mesh,
        scratch_types=[
            pltpu.VMEM((bpw, s), jnp.int32),
            pltpu.VMEM((s, d), jnp.float32),
            pltpu.VMEM((s, d), jnp.float32),
            pltpu.SemaphoreType.DMA,
        ],
        compiler_params=pltpu.CompilerParams(use_tc_tiling_on_sc=False),
    )
    def k(idx_hbm, table_hbm, out_hbm, idx_v, buf0, buf1, sem):
        wid = lax.axis_index("s") * _NC + lax.axis_index("c")
        base = wid * bpw
        pltpu.sync_copy(idx_hbm.at[wid], idx_v)
        bufs = (buf0, buf1)
        pending = pltpu.async_copy(table_hbm.at[idx_v.at[0]], buf0, sem)
        for c in range(bpw):
            cur = bufs[c % 2]
            pending.wait()
            if c + 1 < bpw:
                pending = pltpu.async_copy(
                    table_hbm.at[idx_v.at[c + 1]], bufs[(c + 1) % 2], sem
                )
            pltpu.sync_copy(cur, out_hbm.at[base + c])

    return k(idx_grp, table)


def kernel(input, embedding_matrix):
    b, s = input.shape
    idx_grp = input.reshape(_NW, b // _NW, s).astype(jnp.int32)
    out = _sc_embedding_gather(idx_grp, embedding_matrix)
    return out


# speedup vs baseline: 2.0541x; 1.9989x over previous
import functools

import jax
import jax.numpy as jnp
from jax import lax
from jax.experimental import pallas as pl
from jax.experimental.pallas import tpu as pltpu
from jax.experimental.pallas import tpu_sc as plsc

_NC = 2
_NS = 16
_NW = _NC * _NS


@jax.jit
def _sc_embedding_gather(idx_grp, table_pad):
    nw, bpw, s = idx_grp.shape
    dp = table_pad.shape[1]
    mesh = plsc.VectorSubcoreMesh(core_axis_name="c", subcore_axis_name="s")

    @functools.partial(
        pl.kernel,
        out_type=jax.ShapeDtypeStruct((nw * bpw, s, dp), jnp.float32),
        mesh=mesh,
        scratch_types=[
            pltpu.VMEM((bpw, s), jnp.int32),
            pltpu.VMEM((s, dp), jnp.float32),
            pltpu.VMEM((s, dp), jnp.float32),
            pltpu.SemaphoreType.DMA,
        ],
    )
    def k(idx_hbm, table_hbm, out_hbm, idx_v, buf0, buf1, sem):
        wid = lax.axis_index("s") * _NC + lax.axis_index("c")
        base = wid * bpw
        pltpu.sync_copy(idx_hbm.at[wid], idx_v)
        bufs = (buf0, buf1)
        pending = pltpu.async_copy(table_hbm.at[idx_v.at[0]], buf0, sem)
        for c in range(bpw):
            cur = bufs[c % 2]
            pending.wait()
            if c + 1 < bpw:
                pending = pltpu.async_copy(
                    table_hbm.at[idx_v.at[c + 1]], bufs[(c + 1) % 2], sem
                )
            pltpu.sync_copy(cur, out_hbm.at[base + c])

    return k(idx_grp, table_pad)


def kernel(input, embedding_matrix):
    b, s = input.shape
    v, d = embedding_matrix.shape
    dp = (d + 127) // 128 * 128
    idx_grp = input.reshape(_NW, b // _NW, s).astype(jnp.int32)
    table_pad = jnp.pad(embedding_matrix, ((0, 0), (0, dp - d)))
    out = _sc_embedding_gather(idx_grp, table_pad)
    return out[:, :, :d]
